# baseline (speedup 1.0000x reference)
import functools
import jax
import jax.numpy as jnp
from jax import lax
from jax.experimental import pallas as pl
from jax.experimental.pallas import tpu as pltpu
from jax.experimental.pallas import tpu_sc as plsc

N_CLS = 19
LB_POS = 1.0 - 0.1
LB_NEG = 0.1 / N_CLS
IGN = 255

_NC = 2
_L = 16
_N = 8
_H = 512
_W = 512
_QR = _H // 4
_CR = 8
_CW = 256
_T = (_QR // _CR) * (_W // _CW)


def _sc_body(lab_hbm, out_hbm, lab_v, obuf, isem0, isem1, osem0, osem1):
    cid = lax.axis_index("c")
    sid = lax.axis_index("s")
    wid = sid * _NC + cid
    n = wid // 4
    q = wid % 4
    lrow0 = n * _H + q * _QR
    orow0 = n * N_CLS
    isems = (isem0, isem1)
    osems = (osem0, osem1)

    def in_desc(k, b):
        hrow = (k // 2) * _CR
        c0 = (k % 2) * _CW
        return pltpu.make_async_copy(
            lab_hbm.at[pl.ds(lrow0 + hrow, _CR), pl.ds(c0, _CW)],
            lab_v.at[b], isems[b])

    def out_desc(k, b):
        hrow = q * _QR + (k // 2) * _CR
        c0 = (k % 2) * _CW
        return pltpu.make_async_copy(
            obuf.at[b],
            out_hbm.at[pl.ds(orow0, N_CLS), pl.ds(hrow, _CR), pl.ds(c0, _CW)],
            osems[b])

    def compute(b):
        for r in range(_CR):
            def it(j, carry, r=r):
                for u in range(2):
                    lv = lab_v[b, r, pl.ds(j * 2 * _L + u * _L, _L)]
                    neg = jnp.where(lv == IGN, 0.0, LB_NEG)
                    for c in range(N_CLS):
                        obuf[b, c, r, pl.ds(j * 2 * _L + u * _L, _L)] = (
                            jnp.where(lv == c, LB_POS, neg))
                return carry
            lax.fori_loop(0, _CW // (2 * _L), it, 0)

    in_desc(0, 0).start()
    in_desc(1, 1).start()

    def step(kk, carry):
        for b in (0, 1):
            k = 2 * kk + b
            in_desc(k, b).wait()

            @pl.when(kk >= 1)
            def _(k=k, b=b):
                out_desc(k - 2, b).wait()

            compute(b)
            out_desc(k, b).start()

            @pl.when(k + 2 < _T)
            def _(k=k, b=b):
                in_desc(k + 2, b).start()
        return carry

    lax.fori_loop(0, _T // 2, step, 0)
    out_desc(_T - 2, (_T - 2) % 2).wait()
    out_desc(_T - 1, (_T - 1) % 2).wait()


@functools.partial(
    pl.kernel,
    mesh=plsc.VectorSubcoreMesh(core_axis_name="c", subcore_axis_name="s"),
    out_type=jax.ShapeDtypeStruct((_N * N_CLS, _H, _W), jnp.float32),
    scratch_types=[
        pltpu.VMEM((2, _CR, _CW), jnp.int32),
        pltpu.VMEM((2, N_CLS, _CR, _CW), jnp.float32),
        pltpu.SemaphoreType.DMA,
        pltpu.SemaphoreType.DMA,
        pltpu.SemaphoreType.DMA,
        pltpu.SemaphoreType.DMA,
    ],
)
def _sc_kernel(lab_hbm, out_hbm, lab_v, obuf, isem0, isem1, osem0, osem1):
    _sc_body(lab_hbm, out_hbm, lab_v, obuf, isem0, isem1, osem0, osem1)


def kernel(label):
    n, h, w = label.shape
    out = _sc_kernel(label.reshape(n * h, w))
    return out.reshape(n, N_CLS, h, w)


# speedup vs baseline: 1.3114x; 1.3114x over previous
import functools
import jax
import jax.numpy as jnp
from jax import lax
from jax.experimental import pallas as pl
from jax.experimental.pallas import tpu as pltpu
from jax.experimental.pallas import tpu_sc as plsc

N_CLS = 19
LB_POS = 1.0 - 0.1
LB_NEG = 0.1 / N_CLS
IGN = 255

_NC = 2
_L = 16
_N = 8
_H = 512
_W = 512
_QR = _H // 4
_CR = 8
_CW = 256
_T = (_QR // _CR) * (_W // _CW)


def _sc_body(lab_hbm, out_hbm, lab_v, obuf, isem0, isem1, osem0, osem1):
    cid = lax.axis_index("c")
    sid = lax.axis_index("s")
    wid = sid * _NC + cid
    n = wid // 4
    q = wid % 4
    lrow0 = n * _H + q * _QR
    orow0 = n * N_CLS
    isems = (isem0, isem1)
    osems = (osem0, osem1)

    def in_desc(k, b):
        hrow = (k // 2) * _CR
        c0 = (k % 2) * _CW
        return pltpu.make_async_copy(
            lab_hbm.at[pl.ds(lrow0 + hrow, _CR), pl.ds(c0, _CW)],
            lab_v.at[b], isems[b])

    def out_desc(k, b):
        hrow = q * _QR + (k // 2) * _CR
        c0 = (k % 2) * _CW
        return pltpu.make_async_copy(
            obuf.at[b],
            out_hbm.at[pl.ds(orow0, N_CLS), pl.ds(hrow, _CR), pl.ds(c0, _CW)],
            osems[b])

    def compute(b):
        jl = _CW // _L

        def it(i, carry):
            r = i // jl
            col = (i % jl) * _L
            lv = lab_v[b, r, pl.ds(col, _L)]
            neg = jnp.where(lv == IGN, 0.0, LB_NEG)
            for c in range(N_CLS):
                obuf[b, c, r, pl.ds(col, _L)] = jnp.where(lv == c, LB_POS, neg)
            return carry
        lax.fori_loop(0, _CR * jl, it, 0)

    in_desc(0, 0).start()
    in_desc(1, 1).start()

    def step(kk, carry):
        for b in (0, 1):
            k = 2 * kk + b
            in_desc(k, b).wait()

            @pl.when(kk >= 1)
            def _(k=k, b=b):
                out_desc(k - 2, b).wait()

            compute(b)
            out_desc(k, b).start()

            @pl.when(k + 2 < _T)
            def _(k=k, b=b):
                in_desc(k + 2, b).start()
        return carry

    lax.fori_loop(0, _T // 2, step, 0)
    out_desc(_T - 2, (_T - 2) % 2).wait()
    out_desc(_T - 1, (_T - 1) % 2).wait()


@functools.partial(
    pl.kernel,
    mesh=plsc.VectorSubcoreMesh(core_axis_name="c", subcore_axis_name="s"),
    out_type=jax.ShapeDtypeStruct((_N * N_CLS, _H, _W), jnp.float32),
    scratch_types=[
        pltpu.VMEM((2, _CR, _CW), jnp.int32),
        pltpu.VMEM((2, N_CLS, _CR, _CW), jnp.float32),
        pltpu.SemaphoreType.DMA,
        pltpu.SemaphoreType.DMA,
        pltpu.SemaphoreType.DMA,
        pltpu.SemaphoreType.DMA,
    ],
)
def _sc_kernel(lab_hbm, out_hbm, lab_v, obuf, isem0, isem1, osem0, osem1):
    _sc_body(lab_hbm, out_hbm, lab_v, obuf, isem0, isem1, osem0, osem1)


def kernel(label):
    n, h, w = label.shape
    out = _sc_kernel(label.reshape(n * h, w))
    return out.reshape(n, N_CLS, h, w)
